# baseline (speedup 1.0000x reference)
import jax
import jax.numpy as jnp
import numpy as np
from jax.experimental import pallas as pl
from jax.experimental.pallas import tpu as pltpu

_INV_EM1 = float(1.0 / (np.exp(1.0) - 1.0))
_L = 128
_K = 2048


def _vel_kernel(pos_ref, vel_ref, velout_ref, gamma_ref,
                sx, sy, svx, svy, sem_in, sem_out):
    cps = [
        pltpu.make_async_copy(pos_ref.at[:, 0, :], sx, sem_in),
        pltpu.make_async_copy(pos_ref.at[:, 1, :], sy, sem_in),
        pltpu.make_async_copy(vel_ref.at[:, 0, :], svx, sem_in),
        pltpu.make_async_copy(vel_ref.at[:, 1, :], svy, sem_in),
    ]
    for cp in cps:
        cp.start()
    for cp in cps:
        cp.wait()
    x = sx[...]
    y = sy[...]
    vx = svx[...]
    vy = svy[...]
    m = (x >= 0.0) & (x <= 0.25) & (y >= 0.0) & (y <= 1.0)
    xr = jnp.clip(x * 4.0, 0.0, 1.0)
    t = jnp.exp(jnp.log(xr) * 3.5)
    g = (jnp.exp(t) - 1.0) * _INV_EM1
    g = jnp.minimum(g, 1.0)
    svx[...] = jnp.where(m, vx + g * (1.0 - vx), vx)
    svy[...] = jnp.where(m, vy * (1.0 - g), vy)
    gamma_ref[...] = g.reshape(_K * _L)
    ocs = [
        pltpu.make_async_copy(svx, velout_ref.at[:, 0, :], sem_out),
        pltpu.make_async_copy(svy, velout_ref.at[:, 1, :], sem_out),
    ]
    for cp in ocs:
        cp.start()
    for cp in ocs:
        cp.wait()


def kernel(fluidPosition, fluidVelocity, fluidArea):
    n = fluidPosition.shape[0]
    nk = n // _L
    pos = fluidPosition.reshape(nk, _L, 2).swapaxes(1, 2)
    vel = fluidVelocity.reshape(nk, _L, 2).swapaxes(1, 2)
    grid = (nk + _K - 1) // _K
    vel_out, gamma = pl.pallas_call(
        _vel_kernel,
        grid=(grid,),
        in_specs=[
            pl.BlockSpec((_K, 2, _L), lambda i: (i, 0, 0)),
            pl.BlockSpec((_K, 2, _L), lambda i: (i, 0, 0)),
        ],
        out_specs=[
            pl.BlockSpec((_K, 2, _L), lambda i: (i, 0, 0)),
            pl.BlockSpec((_K * _L,), lambda i: (i,)),
        ],
        out_shape=[
            jax.ShapeDtypeStruct((nk, 2, _L), jnp.float32),
            jax.ShapeDtypeStruct((n,), jnp.float32),
        ],
        scratch_shapes=[
            pltpu.VMEM((_K, _L), jnp.float32),
            pltpu.VMEM((_K, _L), jnp.float32),
            pltpu.VMEM((_K, _L), jnp.float32),
            pltpu.VMEM((_K, _L), jnp.float32),
            pltpu.SemaphoreType.DMA,
            pltpu.SemaphoreType.DMA,
        ],
    )(pos, vel)
    vel_out = vel_out.swapaxes(1, 2).reshape(n, 2)
    return vel_out, gamma


# speedup vs baseline: 1.0265x; 1.0265x over previous
import jax
import jax.numpy as jnp
import numpy as np
from jax.experimental import pallas as pl
from jax.experimental.pallas import tpu as pltpu

_INV_EM1 = float(1.0 / (np.exp(1.0) - 1.0))
_L = 128
_K = 1024


_H = _K // 2


def _vel_kernel(pos_ref, vel_ref, velout_ref, gamma_ref,
                sx, sy, svx, svy, sem_a, sem_b, sem_out):
    halves = []
    for h, sem in ((0, sem_a), (1, sem_b)):
        rows = pl.ds(h * _H, _H)
        cps = [
            pltpu.make_async_copy(pos_ref.at[rows, 0, :], sx.at[rows], sem),
            pltpu.make_async_copy(pos_ref.at[rows, 1, :], sy.at[rows], sem),
            pltpu.make_async_copy(vel_ref.at[rows, 0, :], svx.at[rows], sem),
            pltpu.make_async_copy(vel_ref.at[rows, 1, :], svy.at[rows], sem),
        ]
        for cp in cps:
            cp.start()
        halves.append(cps)
    ocs = []
    for h, cps in enumerate(halves):
        rows = pl.ds(h * _H, _H)
        for cp in cps:
            cp.wait()
        x = sx[rows]
        y = sy[rows]
        vx = svx[rows]
        vy = svy[rows]
        m = (x >= 0.0) & (x <= 0.25) & (y >= 0.0) & (y <= 1.0)
        xr = jnp.clip(x * 4.0, 0.0, 1.0)
        t = jnp.exp(jnp.log(xr) * 3.5)
        g = (jnp.exp(t) - 1.0) * _INV_EM1
        g = jnp.minimum(g, 1.0)
        svx[rows] = jnp.where(m, vx + g * (1.0 - vx), vx)
        svy[rows] = jnp.where(m, vy * (1.0 - g), vy)
        gamma_ref[pl.ds(h * _H * _L, _H * _L)] = g.reshape(_H * _L)
        out_cps = [
            pltpu.make_async_copy(svx.at[rows], velout_ref.at[rows, 0, :], sem_out),
            pltpu.make_async_copy(svy.at[rows], velout_ref.at[rows, 1, :], sem_out),
        ]
        for cp in out_cps:
            cp.start()
        ocs.extend(out_cps)
    for cp in ocs:
        cp.wait()


def kernel(fluidPosition, fluidVelocity, fluidArea):
    n = fluidPosition.shape[0]
    nk = n // _L
    pos = fluidPosition.reshape(nk, _L, 2).swapaxes(1, 2)
    vel = fluidVelocity.reshape(nk, _L, 2).swapaxes(1, 2)
    grid = (nk + _K - 1) // _K
    vel_out, gamma = pl.pallas_call(
        _vel_kernel,
        grid=(grid,),
        in_specs=[
            pl.BlockSpec((_K, 2, _L), lambda i: (i, 0, 0)),
            pl.BlockSpec((_K, 2, _L), lambda i: (i, 0, 0)),
        ],
        out_specs=[
            pl.BlockSpec((_K, 2, _L), lambda i: (i, 0, 0)),
            pl.BlockSpec((_K * _L,), lambda i: (i,)),
        ],
        out_shape=[
            jax.ShapeDtypeStruct((nk, 2, _L), jnp.float32),
            jax.ShapeDtypeStruct((n,), jnp.float32),
        ],
        scratch_shapes=[
            pltpu.VMEM((_K, _L), jnp.float32),
            pltpu.VMEM((_K, _L), jnp.float32),
            pltpu.VMEM((_K, _L), jnp.float32),
            pltpu.VMEM((_K, _L), jnp.float32),
            pltpu.SemaphoreType.DMA,
            pltpu.SemaphoreType.DMA,
            pltpu.SemaphoreType.DMA,
        ],
    )(pos, vel)
    vel_out = vel_out.swapaxes(1, 2).reshape(n, 2)
    return vel_out, gamma
